# Initial kernel scaffold; baseline (speedup 1.0000x reference)
import functools

import jax
import jax.numpy as jnp
from jax import lax
from jax.experimental import pallas as pl
from jax.experimental.pallas import tpu as pltpu
from jax.experimental.pallas import tpu_sc as plsc

VOCAB = 100000
MAX_POS = 8192
D_MODEL = 768
BATCH = 4
SEQ = 2048

_info = plsc.get_sparse_core_info()
NC, NS = _info.num_cores, _info.num_subcores
NW = NC * NS

N_ROWS = BATCH * SEQ
ROWS_PER_W = N_ROWS // NW
CHUNK = 64
NCH = ROWS_PER_W // CHUNK


def _emb_body(ids_hbm, tok_hbm, pos_hbm, out_hbm, idx_v, buf, sem):
    wid = lax.axis_index("s") * NC + lax.axis_index("c")
    base = wid * ROWS_PER_W
    s0 = (wid % (SEQ // ROWS_PER_W)) * ROWS_PER_W
    for j in range(NCH):
        pltpu.sync_copy(ids_hbm.at[pl.ds(base + j * CHUNK, CHUNK)], idx_v.at[j])
        pltpu.sync_copy(pos_hbm.at[pl.ds(s0 + j * CHUNK, CHUNK)], buf)
        pltpu.async_copy(tok_hbm.at[idx_v.at[j]], buf, sem, add=True).wait()
        pltpu.sync_copy(buf, out_hbm.at[pl.ds(base + j * CHUNK, CHUNK)])


_emb = functools.partial(
    pl.kernel,
    out_type=jax.ShapeDtypeStruct((N_ROWS, D_MODEL), jnp.float32),
    mesh=plsc.VectorSubcoreMesh(core_axis_name="c", subcore_axis_name="s"),
    scratch_types=[
        pltpu.VMEM((NCH, CHUNK), jnp.int32),
        pltpu.VMEM((CHUNK, D_MODEL), jnp.float32),
        pltpu.SemaphoreType.DMA,
    ],
)(_emb_body)


@jax.jit
def kernel(input_ids, token_table, pos_table):
    ids_flat = input_ids.reshape(-1).astype(jnp.int32)
    out = _emb(ids_flat, token_table, pos_table)
    return out.reshape(BATCH, SEQ, D_MODEL)


# speedup vs baseline: 1.1447x; 1.1447x over previous
import functools

import jax
import jax.numpy as jnp
from jax import lax
from jax.experimental import pallas as pl
from jax.experimental.pallas import tpu as pltpu
from jax.experimental.pallas import tpu_sc as plsc

VOCAB = 100000
MAX_POS = 8192
D_MODEL = 768
BATCH = 4
SEQ = 2048

_info = plsc.get_sparse_core_info()
NC, NS, NL = _info.num_cores, _info.num_subcores, _info.num_lanes
NW = NC * NS
S_PER_W = SEQ // NW
VPR = D_MODEL // NL


def _emb_body(ids_hbm, tok_hbm, pos_hbm, out_hbm, idx_v, tok_buf, pos_buf, sem):
    wid = lax.axis_index("s") * NC + lax.axis_index("c")
    s0 = wid * S_PER_W
    pltpu.sync_copy(pos_hbm.at[pl.ds(s0, S_PER_W)], pos_buf)
    for b in range(BATCH):
        pltpu.sync_copy(ids_hbm.at[pl.ds(b * SEQ + s0, S_PER_W)], idx_v)
        pltpu.async_copy(tok_hbm.at[idx_v], tok_buf, sem).wait()

        def row(r, _):
            for k in range(VPR):
                x = pos_buf[r, pl.ds(k * NL, NL)]
                plsc.addupdate(tok_buf.at[r, pl.ds(k * NL, NL)], x)
            return 0

        lax.fori_loop(0, S_PER_W, row, 0)
        pltpu.sync_copy(tok_buf, out_hbm.at[pl.ds(b * SEQ + s0, S_PER_W)])


_emb = functools.partial(
    pl.kernel,
    out_type=jax.ShapeDtypeStruct((BATCH * SEQ, D_MODEL), jnp.float32),
    mesh=plsc.VectorSubcoreMesh(core_axis_name="c", subcore_axis_name="s"),
    scratch_types=[
        pltpu.VMEM((S_PER_W,), jnp.int32),
        pltpu.VMEM((S_PER_W, D_MODEL), jnp.float32),
        pltpu.VMEM((S_PER_W, D_MODEL), jnp.float32),
        pltpu.SemaphoreType.DMA,
    ],
)(_emb_body)


@jax.jit
def kernel(input_ids, token_table, pos_table):
    ids_flat = input_ids.reshape(-1).astype(jnp.int32)
    out = _emb(ids_flat, token_table, pos_table)
    return out.reshape(BATCH, SEQ, D_MODEL)
